# baseline (speedup 1.0000x reference)
import functools

import jax
import jax.numpy as jnp
from jax import lax
from jax.experimental import pallas as pl
from jax.experimental.pallas import tpu as pltpu
from jax.experimental.pallas import tpu_sc as plsc

N = 10000
E = 320000
D = 128
NC = 2
NS = 16
NW = NC * NS
K = 64
NCHUNK = 160
EPT = K * NCHUNK
NSLOT = 5
NAGG = N + NS
ROWS_PT = 640
ROWS_LAST = N - 15 * ROWS_PT
CB = 768
CPT = CB // NW


def _agg_body(h_hbm, src_hbm, dst_hbm, zeros_hbm, out_hbm,
              sbufs, dbufs, rbufs, agg, ssems, dsems, gsems, csems):
    c = lax.axis_index("c")
    s = lax.axis_index("s")
    wid = s * NC + c

    row0 = pl.multiple_of(s * ROWS_PT, 8)

    @pl.when(s < NS - 1)
    def _():
        pltpu.sync_copy(zeros_hbm.at[pl.ds(row0, ROWS_PT)],
                        agg.at[pl.ds(row0, ROWS_PT)])

    @pl.when(s == NS - 1)
    def _():
        pltpu.sync_copy(zeros_hbm.at[pl.ds(row0, ROWS_LAST)],
                        agg.at[pl.ds(row0, ROWS_LAST)])

    plsc.subcore_barrier()
    ebase = pl.multiple_of(wid * EPT, 8)

    def idx_start(j, p):
        off = pl.multiple_of(ebase + j * K, 8)
        pltpu.async_copy(src_hbm.at[pl.ds(off, K)], sbufs[p], ssems[p])
        pltpu.async_copy(dst_hbm.at[pl.ds(off, K)], dbufs[p], dsems[p])

    def idx_wait(sem, buf):
        pltpu.make_async_copy(src_hbm.at[pl.ds(0, K)], buf, sem).wait()

    for p in range(NSLOT):
        idx_start(p, p)

    @pl.loop(0, NCHUNK, step=NSLOT)
    def _(g):
        descs = []
        for b in range(NSLOT):
            idx_wait(ssems[b], sbufs[b])
            descs.append(pltpu.async_copy(h_hbm.at[sbufs[b]], rbufs[b],
                                          gsems[b]))
        sdescs = []
        for b in range(NSLOT):
            descs[b].wait()
            idx_wait(dsems[b], dbufs[b])
            sdescs.append(pltpu.async_copy(rbufs[b], agg.at[dbufs[b]],
                                           csems[b], add=True))
        for b in range(NSLOT):
            sdescs[b].wait()

            @pl.when(g + b + NSLOT < NCHUNK)
            def _():
                idx_start(g + b + NSLOT, b)

    plsc.subcore_barrier()

    obase = pl.multiple_of(c * N + s * ROWS_PT, 8)

    @pl.when(s < NS - 1)
    def _():
        pltpu.sync_copy(agg.at[pl.ds(row0, ROWS_PT)],
                        out_hbm.at[pl.ds(obase, ROWS_PT)])

    @pl.when(s == NS - 1)
    def _():
        pltpu.sync_copy(agg.at[pl.ds(row0, ROWS_LAST)],
                        out_hbm.at[pl.ds(obase, ROWS_LAST)])


_agg_call = pl.kernel(
    _agg_body,
    out_type=jax.ShapeDtypeStruct((NC * N, D), jnp.float32),
    mesh=plsc.VectorSubcoreMesh(core_axis_name="c", subcore_axis_name="s"),
    scratch_types=[
        [pltpu.VMEM((K,), jnp.int32)] * NSLOT,
        [pltpu.VMEM((K,), jnp.int32)] * NSLOT,
        [pltpu.VMEM((K, D), jnp.float32)] * NSLOT,
        pltpu.VMEM_SHARED((NAGG, D), jnp.float32),
        [pltpu.SemaphoreType.DMA] * NSLOT,
        [pltpu.SemaphoreType.DMA] * NSLOT,
        [pltpu.SemaphoreType.DMA] * NSLOT,
        [pltpu.SemaphoreType.DMA] * NSLOT,
    ],
)


def _gather_body(h_hbm, idx_hbm, out_hbm, idx_v, rows_v, sem):
    c = lax.axis_index("c")
    s = lax.axis_index("s")
    wid = s * NC + c
    base = pl.multiple_of(wid * CPT, 8)
    pltpu.sync_copy(idx_hbm.at[pl.ds(base, CPT)], idx_v)
    pltpu.async_copy(h_hbm.at[idx_v], rows_v, sem).wait()
    pltpu.sync_copy(rows_v, out_hbm.at[pl.ds(base, CPT)])


_gather_call = pl.kernel(
    _gather_body,
    out_type=jax.ShapeDtypeStruct((CB, D), jnp.float32),
    mesh=plsc.VectorSubcoreMesh(core_axis_name="c", subcore_axis_name="s"),
    scratch_types=[
        pltpu.VMEM((CPT,), jnp.int32),
        pltpu.VMEM((CPT, D), jnp.float32),
        pltpu.SemaphoreType.DMA,
    ],
)


_MLP_BLK = 1000


def _mlp_body(h_ref, p0_ref, p1_ref, w1_ref, b1_ref, w2_ref, b2_ref, o_ref):
    z = h_ref[...] + p0_ref[...] + p1_ref[...]
    a = jnp.dot(z, w1_ref[...], preferred_element_type=jnp.float32) + b1_ref[...]
    a = jnp.maximum(a, 0.0)
    o_ref[...] = (jnp.dot(a, w2_ref[...], preferred_element_type=jnp.float32)
                  + b2_ref[...])


def _mlp(h, p0, p1, w1, b1, w2, b2):
    grid = (N // _MLP_BLK,)
    blk = lambda i: (i, 0)
    zero = lambda i: (0, 0)
    return pl.pallas_call(
        _mlp_body,
        grid=grid,
        in_specs=[
            pl.BlockSpec((_MLP_BLK, D), blk),
            pl.BlockSpec((_MLP_BLK, D), blk),
            pl.BlockSpec((_MLP_BLK, D), blk),
            pl.BlockSpec((D, D), zero),
            pl.BlockSpec((1, D), zero),
            pl.BlockSpec((D, D), zero),
            pl.BlockSpec((1, D), zero),
        ],
        out_specs=pl.BlockSpec((_MLP_BLK, D), blk),
        out_shape=jax.ShapeDtypeStruct((N, D), jnp.float32),
    )(h, p0, p1, w1, b1.reshape(1, D), w2, b2.reshape(1, D))


def _score_body(tt_ref, g_ref, tab_ref, wp1_ref, bp1_ref, wp2_ref, bp2_ref,
                o_ref):
    U = g_ref[0:256, :]
    V = g_ref[256:512, :]
    fu = g_ref[512:513, :]
    fv = g_ref[513:514, :]
    S = U + V
    A = jnp.abs(U - V)
    ffs = fu + fv
    ffd = jnp.abs(fu - fv)
    ti = tt_ref[0]
    temb = tab_ref[pl.ds(ti, 1), :]
    dot = lambda a, b: jnp.dot(a, b, preferred_element_type=jnp.float32)
    base = (dot(ffs, wp1_ref[0:128, :]) + dot(ffd, wp1_ref[128:256, :])
            + dot(temb, wp1_ref[512:640, :]) + bp1_ref[...])
    hid = jnp.maximum(
        dot(S, wp1_ref[256:384, :]) + dot(A, wp1_ref[384:512, :]) + base, 0.0)
    o_ref[...] = (jnp.sum(hid * wp2_ref[...], axis=1, keepdims=True)
                  + bp2_ref[...])


def _score(tt, gat, t_table, wp1, bp1, wp2, bp2):
    return pl.pallas_call(
        _score_body,
        in_specs=[
            pl.BlockSpec(memory_space=pltpu.SMEM),
            pl.BlockSpec(memory_space=pltpu.VMEM),
            pl.BlockSpec(memory_space=pltpu.VMEM),
            pl.BlockSpec(memory_space=pltpu.VMEM),
            pl.BlockSpec(memory_space=pltpu.VMEM),
            pl.BlockSpec(memory_space=pltpu.VMEM),
            pl.BlockSpec(memory_space=pltpu.VMEM),
        ],
        out_shape=jax.ShapeDtypeStruct((256, 1), jnp.float32),
    )(tt, gat, t_table, wp1, bp1.reshape(1, D), wp2.reshape(1, D),
      bp2.reshape(1, 1))


def kernel(x, edge_index, first_edge, candidate_edges, t,
           w1_0, b1_0, w2_0, b2_0, w1_1, b1_1, w2_1, b2_1,
           w1_2, b1_2, w2_2, b2_2, wp1, bp1, wp2, bp2, t_table):
    real = E // NW
    pad = EPT - real
    dummy = N + (jnp.arange(NW, dtype=edge_index.dtype) // NC)[:, None]
    src = jnp.concatenate([
        edge_index[0].reshape(NW, real),
        jnp.zeros((NW, pad), edge_index.dtype)], axis=1).reshape(-1)
    dst = jnp.concatenate([
        edge_index[1].reshape(NW, real),
        jnp.broadcast_to(dummy, (NW, pad))], axis=1).reshape(-1)
    zeros = jnp.zeros((N, D), jnp.float32)

    h = x
    for (w1, b1, w2, b2) in ((w1_0, b1_0, w2_0, b2_0),
                             (w1_1, b1_1, w2_1, b2_1),
                             (w1_2, b1_2, w2_2, b2_2)):
        parts = _agg_call(h, src, dst, zeros)
        h = _mlp(h, parts[:N], parts[N:], w1, b1, w2, b2)

    gidx = jnp.concatenate([
        candidate_edges[:, 0], candidate_edges[:, 1], first_edge,
        jnp.zeros((CB - 2 * 256 - 2,), jnp.int32),
    ]).astype(jnp.int32)
    gat = _gather_call(h, gidx)

    tt = jnp.clip(t, 0, t_table.shape[0] - 1).astype(jnp.int32).reshape(1)
    scores = _score(tt, gat, t_table, wp1, bp1, wp2, bp2)
    return scores[:, 0]


# speedup vs baseline: 1.0004x; 1.0004x over previous
import functools

import jax
import jax.numpy as jnp
from jax import lax
from jax.experimental import pallas as pl
from jax.experimental.pallas import tpu as pltpu
from jax.experimental.pallas import tpu_sc as plsc

N = 10000
E = 320000
D = 128
NC = 2
NS = 16
NW = NC * NS
K = 64
NCHUNK = 160
EPT = K * NCHUNK
NSLOT = 5
NAGG = N + NS * 16
ROWS_PT = 640
ROWS_LAST = N - 15 * ROWS_PT
CB = 768
CPT = CB // NW


def _agg_body(h_hbm, src_hbm, dst_hbm, zeros_hbm, out_hbm,
              sbufs, dbufs, rbufs, agg, ssems, dsems, gsems, csems):
    c = lax.axis_index("c")
    s = lax.axis_index("s")
    wid = s * NC + c

    row0 = pl.multiple_of(s * ROWS_PT, 8)

    @pl.when(s < NS - 1)
    def _():
        pltpu.sync_copy(zeros_hbm.at[pl.ds(row0, ROWS_PT)],
                        agg.at[pl.ds(row0, ROWS_PT)])

    @pl.when(s == NS - 1)
    def _():
        pltpu.sync_copy(zeros_hbm.at[pl.ds(row0, ROWS_LAST)],
                        agg.at[pl.ds(row0, ROWS_LAST)])

    plsc.subcore_barrier()
    ebase = pl.multiple_of(wid * EPT, 8)

    def idx_start(j, p):
        off = pl.multiple_of(ebase + j * K, 8)
        pltpu.async_copy(src_hbm.at[pl.ds(off, K)], sbufs[p], ssems[p])
        pltpu.async_copy(dst_hbm.at[pl.ds(off, K)], dbufs[p], dsems[p])

    def idx_wait(sem, buf):
        pltpu.make_async_copy(src_hbm.at[pl.ds(0, K)], buf, sem).wait()

    for p in range(NSLOT):
        idx_start(p, p)

    @pl.loop(0, NCHUNK, step=NSLOT)
    def _(g):
        descs = []
        for b in range(NSLOT):
            idx_wait(ssems[b], sbufs[b])
            descs.append(pltpu.async_copy(h_hbm.at[sbufs[b]], rbufs[b],
                                          gsems[b]))
        sdescs = []
        for b in range(NSLOT):
            descs[b].wait()
            idx_wait(dsems[b], dbufs[b])
            sdescs.append(pltpu.async_copy(rbufs[b], agg.at[dbufs[b]],
                                           csems[b], add=True))
        for b in range(NSLOT):
            sdescs[b].wait()

            @pl.when(g + b + NSLOT < NCHUNK)
            def _():
                idx_start(g + b + NSLOT, b)

    plsc.subcore_barrier()

    obase = pl.multiple_of(c * N + s * ROWS_PT, 8)

    @pl.when(s < NS - 1)
    def _():
        pltpu.sync_copy(agg.at[pl.ds(row0, ROWS_PT)],
                        out_hbm.at[pl.ds(obase, ROWS_PT)])

    @pl.when(s == NS - 1)
    def _():
        pltpu.sync_copy(agg.at[pl.ds(row0, ROWS_LAST)],
                        out_hbm.at[pl.ds(obase, ROWS_LAST)])


_agg_call = pl.kernel(
    _agg_body,
    out_type=jax.ShapeDtypeStruct((NC * N, D), jnp.float32),
    mesh=plsc.VectorSubcoreMesh(core_axis_name="c", subcore_axis_name="s"),
    scratch_types=[
        [pltpu.VMEM((K,), jnp.int32)] * NSLOT,
        [pltpu.VMEM((K,), jnp.int32)] * NSLOT,
        [pltpu.VMEM((K, D), jnp.float32)] * NSLOT,
        pltpu.VMEM_SHARED((NAGG, D), jnp.float32),
        [pltpu.SemaphoreType.DMA] * NSLOT,
        [pltpu.SemaphoreType.DMA] * NSLOT,
        [pltpu.SemaphoreType.DMA] * NSLOT,
        [pltpu.SemaphoreType.DMA] * NSLOT,
    ],
)


def _gather_body(h_hbm, idx_hbm, out_hbm, idx_v, rows_v, sem):
    c = lax.axis_index("c")
    s = lax.axis_index("s")
    wid = s * NC + c
    base = pl.multiple_of(wid * CPT, 8)
    pltpu.sync_copy(idx_hbm.at[pl.ds(base, CPT)], idx_v)
    pltpu.async_copy(h_hbm.at[idx_v], rows_v, sem).wait()
    pltpu.sync_copy(rows_v, out_hbm.at[pl.ds(base, CPT)])


_gather_call = pl.kernel(
    _gather_body,
    out_type=jax.ShapeDtypeStruct((CB, D), jnp.float32),
    mesh=plsc.VectorSubcoreMesh(core_axis_name="c", subcore_axis_name="s"),
    scratch_types=[
        pltpu.VMEM((CPT,), jnp.int32),
        pltpu.VMEM((CPT, D), jnp.float32),
        pltpu.SemaphoreType.DMA,
    ],
)


_MLP_BLK = 1000


def _mlp_body(h_ref, p0_ref, p1_ref, w1_ref, b1_ref, w2_ref, b2_ref, o_ref):
    z = h_ref[...] + p0_ref[...] + p1_ref[...]
    a = jnp.dot(z, w1_ref[...], preferred_element_type=jnp.float32) + b1_ref[...]
    a = jnp.maximum(a, 0.0)
    o_ref[...] = (jnp.dot(a, w2_ref[...], preferred_element_type=jnp.float32)
                  + b2_ref[...])


def _mlp(h, p0, p1, w1, b1, w2, b2):
    grid = (N // _MLP_BLK,)
    blk = lambda i: (i, 0)
    zero = lambda i: (0, 0)
    return pl.pallas_call(
        _mlp_body,
        grid=grid,
        in_specs=[
            pl.BlockSpec((_MLP_BLK, D), blk),
            pl.BlockSpec((_MLP_BLK, D), blk),
            pl.BlockSpec((_MLP_BLK, D), blk),
            pl.BlockSpec((D, D), zero),
            pl.BlockSpec((1, D), zero),
            pl.BlockSpec((D, D), zero),
            pl.BlockSpec((1, D), zero),
        ],
        out_specs=pl.BlockSpec((_MLP_BLK, D), blk),
        out_shape=jax.ShapeDtypeStruct((N, D), jnp.float32),
    )(h, p0, p1, w1, b1.reshape(1, D), w2, b2.reshape(1, D))


def _score_body(tt_ref, g_ref, tab_ref, wp1_ref, bp1_ref, wp2_ref, bp2_ref,
                o_ref):
    U = g_ref[0:256, :]
    V = g_ref[256:512, :]
    fu = g_ref[512:513, :]
    fv = g_ref[513:514, :]
    S = U + V
    A = jnp.abs(U - V)
    ffs = fu + fv
    ffd = jnp.abs(fu - fv)
    ti = tt_ref[0]
    temb = tab_ref[pl.ds(ti, 1), :]
    dot = lambda a, b: jnp.dot(a, b, preferred_element_type=jnp.float32)
    base = (dot(ffs, wp1_ref[0:128, :]) + dot(ffd, wp1_ref[128:256, :])
            + dot(temb, wp1_ref[512:640, :]) + bp1_ref[...])
    hid = jnp.maximum(
        dot(S, wp1_ref[256:384, :]) + dot(A, wp1_ref[384:512, :]) + base, 0.0)
    o_ref[...] = (jnp.sum(hid * wp2_ref[...], axis=1, keepdims=True)
                  + bp2_ref[...])


def _score(tt, gat, t_table, wp1, bp1, wp2, bp2):
    return pl.pallas_call(
        _score_body,
        in_specs=[
            pl.BlockSpec(memory_space=pltpu.SMEM),
            pl.BlockSpec(memory_space=pltpu.VMEM),
            pl.BlockSpec(memory_space=pltpu.VMEM),
            pl.BlockSpec(memory_space=pltpu.VMEM),
            pl.BlockSpec(memory_space=pltpu.VMEM),
            pl.BlockSpec(memory_space=pltpu.VMEM),
            pl.BlockSpec(memory_space=pltpu.VMEM),
        ],
        out_shape=jax.ShapeDtypeStruct((256, 1), jnp.float32),
    )(tt, gat, t_table, wp1, bp1.reshape(1, D), wp2.reshape(1, D),
      bp2.reshape(1, 1))


def kernel(x, edge_index, first_edge, candidate_edges, t,
           w1_0, b1_0, w2_0, b2_0, w1_1, b1_1, w2_1, b2_1,
           w1_2, b1_2, w2_2, b2_2, wp1, bp1, wp2, bp2, t_table):
    real = E // NW
    pad = EPT - real
    dummy = (N + (jnp.arange(NW, dtype=edge_index.dtype) // NC)[:, None] * 16
             + (jnp.arange(pad, dtype=edge_index.dtype) % 16)[None, :])
    src = jnp.concatenate([
        edge_index[0].reshape(NW, real),
        jnp.zeros((NW, pad), edge_index.dtype)], axis=1).reshape(-1)
    dst = jnp.concatenate([
        edge_index[1].reshape(NW, real),
        dummy], axis=1).reshape(-1)
    zeros = jnp.zeros((N, D), jnp.float32)

    h = x
    for (w1, b1, w2, b2) in ((w1_0, b1_0, w2_0, b2_0),
                             (w1_1, b1_1, w2_1, b2_1),
                             (w1_2, b1_2, w2_2, b2_2)):
        parts = _agg_call(h, src, dst, zeros)
        h = _mlp(h, parts[:N], parts[N:], w1, b1, w2, b2)

    gidx = jnp.concatenate([
        candidate_edges[:, 0], candidate_edges[:, 1], first_edge,
        jnp.zeros((CB - 2 * 256 - 2,), jnp.int32),
    ]).astype(jnp.int32)
    gat = _gather_call(h, gidx)

    tt = jnp.clip(t, 0, t_table.shape[0] - 1).astype(jnp.int32).reshape(1)
    scores = _score(tt, gat, t_table, wp1, bp1, wp2, bp2)
    return scores[:, 0]


# speedup vs baseline: 1.0012x; 1.0008x over previous
import functools

import jax
import jax.numpy as jnp
from jax import lax
from jax.experimental import pallas as pl
from jax.experimental.pallas import tpu as pltpu
from jax.experimental.pallas import tpu_sc as plsc

N = 10000
E = 320000
D = 128
NC = 2
NS = 16
NW = NC * NS
K = 80
NCHUNK = 128
EPT = K * NCHUNK
NSLOT = 4
NAGG = N + NS * 16
ROWS_PT = 640
ROWS_LAST = N - 15 * ROWS_PT
CB = 768
CPT = CB // NW


def _agg_body(h_hbm, src_hbm, dst_hbm, zeros_hbm, out_hbm,
              sbufs, dbufs, rbufs, agg, ssems, dsems, gsems, csems):
    c = lax.axis_index("c")
    s = lax.axis_index("s")
    wid = s * NC + c

    row0 = pl.multiple_of(s * ROWS_PT, 8)

    @pl.when(s < NS - 1)
    def _():
        pltpu.sync_copy(zeros_hbm.at[pl.ds(row0, ROWS_PT)],
                        agg.at[pl.ds(row0, ROWS_PT)])

    @pl.when(s == NS - 1)
    def _():
        pltpu.sync_copy(zeros_hbm.at[pl.ds(row0, ROWS_LAST)],
                        agg.at[pl.ds(row0, ROWS_LAST)])

    plsc.subcore_barrier()
    ebase = pl.multiple_of(wid * EPT, 8)

    def idx_start(j, p):
        off = pl.multiple_of(ebase + j * K, 8)
        pltpu.async_copy(src_hbm.at[pl.ds(off, K)], sbufs[p], ssems[p])
        pltpu.async_copy(dst_hbm.at[pl.ds(off, K)], dbufs[p], dsems[p])

    def idx_wait(sem, buf):
        pltpu.make_async_copy(src_hbm.at[pl.ds(0, K)], buf, sem).wait()

    for p in range(NSLOT):
        idx_start(p, p)

    @pl.loop(0, NCHUNK, step=NSLOT)
    def _(g):
        descs = []
        for b in range(NSLOT):
            idx_wait(ssems[b], sbufs[b])
            descs.append(pltpu.async_copy(h_hbm.at[sbufs[b]], rbufs[b],
                                          gsems[b]))
        sdescs = []
        for b in range(NSLOT):
            descs[b].wait()
            idx_wait(dsems[b], dbufs[b])
            sdescs.append(pltpu.async_copy(rbufs[b], agg.at[dbufs[b]],
                                           csems[b], add=True))
        for b in range(NSLOT):
            sdescs[b].wait()

            @pl.when(g + b + NSLOT < NCHUNK)
            def _():
                idx_start(g + b + NSLOT, b)

    plsc.subcore_barrier()

    obase = pl.multiple_of(c * N + s * ROWS_PT, 8)

    @pl.when(s < NS - 1)
    def _():
        pltpu.sync_copy(agg.at[pl.ds(row0, ROWS_PT)],
                        out_hbm.at[pl.ds(obase, ROWS_PT)])

    @pl.when(s == NS - 1)
    def _():
        pltpu.sync_copy(agg.at[pl.ds(row0, ROWS_LAST)],
                        out_hbm.at[pl.ds(obase, ROWS_LAST)])


_agg_call = pl.kernel(
    _agg_body,
    out_type=jax.ShapeDtypeStruct((NC * N, D), jnp.float32),
    mesh=plsc.VectorSubcoreMesh(core_axis_name="c", subcore_axis_name="s"),
    scratch_types=[
        [pltpu.VMEM((K,), jnp.int32)] * NSLOT,
        [pltpu.VMEM((K,), jnp.int32)] * NSLOT,
        [pltpu.VMEM((K, D), jnp.float32)] * NSLOT,
        pltpu.VMEM_SHARED((NAGG, D), jnp.float32),
        [pltpu.SemaphoreType.DMA] * NSLOT,
        [pltpu.SemaphoreType.DMA] * NSLOT,
        [pltpu.SemaphoreType.DMA] * NSLOT,
        [pltpu.SemaphoreType.DMA] * NSLOT,
    ],
)


def _gather_body(h_hbm, idx_hbm, out_hbm, idx_v, rows_v, sem):
    c = lax.axis_index("c")
    s = lax.axis_index("s")
    wid = s * NC + c
    base = pl.multiple_of(wid * CPT, 8)
    pltpu.sync_copy(idx_hbm.at[pl.ds(base, CPT)], idx_v)
    pltpu.async_copy(h_hbm.at[idx_v], rows_v, sem).wait()
    pltpu.sync_copy(rows_v, out_hbm.at[pl.ds(base, CPT)])


_gather_call = pl.kernel(
    _gather_body,
    out_type=jax.ShapeDtypeStruct((CB, D), jnp.float32),
    mesh=plsc.VectorSubcoreMesh(core_axis_name="c", subcore_axis_name="s"),
    scratch_types=[
        pltpu.VMEM((CPT,), jnp.int32),
        pltpu.VMEM((CPT, D), jnp.float32),
        pltpu.SemaphoreType.DMA,
    ],
)


_MLP_BLK = 1000


def _mlp_body(h_ref, p0_ref, p1_ref, w1_ref, b1_ref, w2_ref, b2_ref, o_ref):
    z = h_ref[...] + p0_ref[...] + p1_ref[...]
    a = jnp.dot(z, w1_ref[...], preferred_element_type=jnp.float32) + b1_ref[...]
    a = jnp.maximum(a, 0.0)
    o_ref[...] = (jnp.dot(a, w2_ref[...], preferred_element_type=jnp.float32)
                  + b2_ref[...])


def _mlp(h, p0, p1, w1, b1, w2, b2):
    grid = (N // _MLP_BLK,)
    blk = lambda i: (i, 0)
    zero = lambda i: (0, 0)
    return pl.pallas_call(
        _mlp_body,
        grid=grid,
        in_specs=[
            pl.BlockSpec((_MLP_BLK, D), blk),
            pl.BlockSpec((_MLP_BLK, D), blk),
            pl.BlockSpec((_MLP_BLK, D), blk),
            pl.BlockSpec((D, D), zero),
            pl.BlockSpec((1, D), zero),
            pl.BlockSpec((D, D), zero),
            pl.BlockSpec((1, D), zero),
        ],
        out_specs=pl.BlockSpec((_MLP_BLK, D), blk),
        out_shape=jax.ShapeDtypeStruct((N, D), jnp.float32),
    )(h, p0, p1, w1, b1.reshape(1, D), w2, b2.reshape(1, D))


def _score_body(tt_ref, g_ref, tab_ref, wp1_ref, bp1_ref, wp2_ref, bp2_ref,
                o_ref):
    U = g_ref[0:256, :]
    V = g_ref[256:512, :]
    fu = g_ref[512:513, :]
    fv = g_ref[513:514, :]
    S = U + V
    A = jnp.abs(U - V)
    ffs = fu + fv
    ffd = jnp.abs(fu - fv)
    ti = tt_ref[0]
    temb = tab_ref[pl.ds(ti, 1), :]
    dot = lambda a, b: jnp.dot(a, b, preferred_element_type=jnp.float32)
    base = (dot(ffs, wp1_ref[0:128, :]) + dot(ffd, wp1_ref[128:256, :])
            + dot(temb, wp1_ref[512:640, :]) + bp1_ref[...])
    hid = jnp.maximum(
        dot(S, wp1_ref[256:384, :]) + dot(A, wp1_ref[384:512, :]) + base, 0.0)
    o_ref[...] = (jnp.sum(hid * wp2_ref[...], axis=1, keepdims=True)
                  + bp2_ref[...])


def _score(tt, gat, t_table, wp1, bp1, wp2, bp2):
    return pl.pallas_call(
        _score_body,
        in_specs=[
            pl.BlockSpec(memory_space=pltpu.SMEM),
            pl.BlockSpec(memory_space=pltpu.VMEM),
            pl.BlockSpec(memory_space=pltpu.VMEM),
            pl.BlockSpec(memory_space=pltpu.VMEM),
            pl.BlockSpec(memory_space=pltpu.VMEM),
            pl.BlockSpec(memory_space=pltpu.VMEM),
            pl.BlockSpec(memory_space=pltpu.VMEM),
        ],
        out_shape=jax.ShapeDtypeStruct((256, 1), jnp.float32),
    )(tt, gat, t_table, wp1, bp1.reshape(1, D), wp2.reshape(1, D),
      bp2.reshape(1, 1))


def kernel(x, edge_index, first_edge, candidate_edges, t,
           w1_0, b1_0, w2_0, b2_0, w1_1, b1_1, w2_1, b2_1,
           w1_2, b1_2, w2_2, b2_2, wp1, bp1, wp2, bp2, t_table):
    real = E // NW
    pad = EPT - real
    dummy = (N + (jnp.arange(NW, dtype=edge_index.dtype) // NC)[:, None] * 16
             + (jnp.arange(pad, dtype=edge_index.dtype) % 16)[None, :])
    src = jnp.concatenate([
        edge_index[0].reshape(NW, real),
        jnp.zeros((NW, pad), edge_index.dtype)], axis=1).reshape(-1)
    dst = jnp.concatenate([
        edge_index[1].reshape(NW, real),
        dummy], axis=1).reshape(-1)
    zeros = jnp.zeros((N, D), jnp.float32)

    h = x
    for (w1, b1, w2, b2) in ((w1_0, b1_0, w2_0, b2_0),
                             (w1_1, b1_1, w2_1, b2_1),
                             (w1_2, b1_2, w2_2, b2_2)):
        parts = _agg_call(h, src, dst, zeros)
        h = _mlp(h, parts[:N], parts[N:], w1, b1, w2, b2)

    gidx = jnp.concatenate([
        candidate_edges[:, 0], candidate_edges[:, 1], first_edge,
        jnp.zeros((CB - 2 * 256 - 2,), jnp.int32),
    ]).astype(jnp.int32)
    gat = _gather_call(h, gidx)

    tt = jnp.clip(t, 0, t_table.shape[0] - 1).astype(jnp.int32).reshape(1)
    scores = _score(tt, gat, t_table, wp1, bp1, wp2, bp2)
    return scores[:, 0]


# speedup vs baseline: 2.6761x; 2.6728x over previous
import functools

import jax
import jax.numpy as jnp
from jax import lax
from jax.experimental import pallas as pl
from jax.experimental.pallas import tpu as pltpu
from jax.experimental.pallas import tpu_sc as plsc

N = 10000
E = 320000
D = 128
NC = 2
NS = 16
NW = NC * NS
EPT = E // NW
K = 80
NCHUNK = EPT // K
NSLOT = 4
NFULL = NCHUNK - 1
ROWS_PT = 640
ROWS_LAST = N - 15 * ROWS_PT
CB = 768
CPT = CB // NW


def _agg_body(h_hbm, src_hbm, dst_hbm, zeros_hbm, out_hbm,
              sbufs, dbufs, rbufs, agg, ssems, dsems, gsems, csems):
    c = lax.axis_index("c")
    s = lax.axis_index("s")
    wid = s * NC + c

    row0 = pl.multiple_of(s * ROWS_PT, 8)

    @pl.when(s < NS - 1)
    def _():
        pltpu.sync_copy(zeros_hbm.at[pl.ds(row0, ROWS_PT)],
                        agg.at[pl.ds(row0, ROWS_PT)])

    @pl.when(s == NS - 1)
    def _():
        pltpu.sync_copy(zeros_hbm.at[pl.ds(row0, ROWS_LAST)],
                        agg.at[pl.ds(row0, ROWS_LAST)])

    plsc.subcore_barrier()
    ebase = pl.multiple_of(wid * EPT, 8)

    def idx_start(j, p):
        off = pl.multiple_of(ebase + j * K, 8)
        pltpu.async_copy(src_hbm.at[pl.ds(off, K)], sbufs[p], ssems[p])
        pltpu.async_copy(dst_hbm.at[pl.ds(off, K)], dbufs[p], dsems[p])

    def idx_wait(sem, buf):
        pltpu.make_async_copy(src_hbm.at[pl.ds(0, K)], buf, sem).wait()

    for p in range(NSLOT):
        idx_start(p, p)

    @pl.loop(0, NFULL, step=NSLOT)
    def _(g):
        descs = []
        for b in range(NSLOT):
            idx_wait(ssems[b], sbufs[b])
            descs.append(pltpu.async_copy(h_hbm.at[sbufs[b]], rbufs[b],
                                          gsems[b]))
        sdescs = []
        for b in range(NSLOT):
            descs[b].wait()
            idx_wait(dsems[b], dbufs[b])
            sdescs.append(pltpu.async_copy(rbufs[b], agg.at[dbufs[b]],
                                           csems[b], add=True))
        for b in range(NSLOT):
            sdescs[b].wait()

            @pl.when(g + b + NSLOT < NCHUNK)
            def _():
                idx_start(g + b + NSLOT, b)

    idx_wait(ssems[0], sbufs[0])
    pltpu.async_copy(h_hbm.at[sbufs[0]], rbufs[0], gsems[0]).wait()
    idx_wait(dsems[0], dbufs[0])
    pltpu.sync_copy(rbufs[0], agg.at[dbufs[0]], add=True)

    plsc.subcore_barrier()

    obase = pl.multiple_of(c * N + s * ROWS_PT, 8)

    @pl.when(s < NS - 1)
    def _():
        pltpu.sync_copy(agg.at[pl.ds(row0, ROWS_PT)],
                        out_hbm.at[pl.ds(obase, ROWS_PT)])

    @pl.when(s == NS - 1)
    def _():
        pltpu.sync_copy(agg.at[pl.ds(row0, ROWS_LAST)],
                        out_hbm.at[pl.ds(obase, ROWS_LAST)])


_agg_call = pl.kernel(
    _agg_body,
    out_type=jax.ShapeDtypeStruct((NC * N, D), jnp.float32),
    mesh=plsc.VectorSubcoreMesh(core_axis_name="c", subcore_axis_name="s"),
    scratch_types=[
        [pltpu.VMEM((K,), jnp.int32)] * NSLOT,
        [pltpu.VMEM((K,), jnp.int32)] * NSLOT,
        [pltpu.VMEM((K, D), jnp.float32)] * NSLOT,
        pltpu.VMEM_SHARED((N + 256, D), jnp.float32),
        [pltpu.SemaphoreType.DMA] * NSLOT,
        [pltpu.SemaphoreType.DMA] * NSLOT,
        [pltpu.SemaphoreType.DMA] * NSLOT,
        [pltpu.SemaphoreType.DMA] * NSLOT,
    ],
)


def _gather_body(h_hbm, idx_hbm, out_hbm, idx_v, rows_v, sem):
    c = lax.axis_index("c")
    s = lax.axis_index("s")
    wid = s * NC + c
    base = pl.multiple_of(wid * CPT, 8)
    pltpu.sync_copy(idx_hbm.at[pl.ds(base, CPT)], idx_v)
    pltpu.async_copy(h_hbm.at[idx_v], rows_v, sem).wait()
    pltpu.sync_copy(rows_v, out_hbm.at[pl.ds(base, CPT)])


_gather_call = pl.kernel(
    _gather_body,
    out_type=jax.ShapeDtypeStruct((CB, D), jnp.float32),
    mesh=plsc.VectorSubcoreMesh(core_axis_name="c", subcore_axis_name="s"),
    scratch_types=[
        pltpu.VMEM((CPT,), jnp.int32),
        pltpu.VMEM((CPT, D), jnp.float32),
        pltpu.SemaphoreType.DMA,
    ],
)


_MLP_BLK = 1000


def _mlp_body(h_ref, p0_ref, p1_ref, w1_ref, b1_ref, w2_ref, b2_ref, o_ref):
    z = h_ref[...] + p0_ref[...] + p1_ref[...]
    a = jnp.dot(z, w1_ref[...], preferred_element_type=jnp.float32) + b1_ref[...]
    a = jnp.maximum(a, 0.0)
    o_ref[...] = (jnp.dot(a, w2_ref[...], preferred_element_type=jnp.float32)
                  + b2_ref[...])


def _mlp(h, p0, p1, w1, b1, w2, b2):
    grid = (N // _MLP_BLK,)
    blk = lambda i: (i, 0)
    zero = lambda i: (0, 0)
    return pl.pallas_call(
        _mlp_body,
        grid=grid,
        in_specs=[
            pl.BlockSpec((_MLP_BLK, D), blk),
            pl.BlockSpec((_MLP_BLK, D), blk),
            pl.BlockSpec((_MLP_BLK, D), blk),
            pl.BlockSpec((D, D), zero),
            pl.BlockSpec((1, D), zero),
            pl.BlockSpec((D, D), zero),
            pl.BlockSpec((1, D), zero),
        ],
        out_specs=pl.BlockSpec((_MLP_BLK, D), blk),
        out_shape=jax.ShapeDtypeStruct((N, D), jnp.float32),
    )(h, p0, p1, w1, b1.reshape(1, D), w2, b2.reshape(1, D))


def _score_body(tt_ref, g_ref, tab_ref, wp1_ref, bp1_ref, wp2_ref, bp2_ref,
                o_ref):
    U = g_ref[0:256, :]
    V = g_ref[256:512, :]
    fu = g_ref[512:513, :]
    fv = g_ref[513:514, :]
    S = U + V
    A = jnp.abs(U - V)
    ffs = fu + fv
    ffd = jnp.abs(fu - fv)
    ti = tt_ref[0]
    temb = tab_ref[pl.ds(ti, 1), :]
    dot = lambda a, b: jnp.dot(a, b, preferred_element_type=jnp.float32)
    base = (dot(ffs, wp1_ref[0:128, :]) + dot(ffd, wp1_ref[128:256, :])
            + dot(temb, wp1_ref[512:640, :]) + bp1_ref[...])
    hid = jnp.maximum(
        dot(S, wp1_ref[256:384, :]) + dot(A, wp1_ref[384:512, :]) + base, 0.0)
    o_ref[...] = (jnp.sum(hid * wp2_ref[...], axis=1, keepdims=True)
                  + bp2_ref[...])


def _score(tt, gat, t_table, wp1, bp1, wp2, bp2):
    return pl.pallas_call(
        _score_body,
        in_specs=[
            pl.BlockSpec(memory_space=pltpu.SMEM),
            pl.BlockSpec(memory_space=pltpu.VMEM),
            pl.BlockSpec(memory_space=pltpu.VMEM),
            pl.BlockSpec(memory_space=pltpu.VMEM),
            pl.BlockSpec(memory_space=pltpu.VMEM),
            pl.BlockSpec(memory_space=pltpu.VMEM),
            pl.BlockSpec(memory_space=pltpu.VMEM),
        ],
        out_shape=jax.ShapeDtypeStruct((256, 1), jnp.float32),
    )(tt, gat, t_table, wp1, bp1.reshape(1, D), wp2.reshape(1, D),
      bp2.reshape(1, 1))


def kernel(x, edge_index, first_edge, candidate_edges, t,
           w1_0, b1_0, w2_0, b2_0, w1_1, b1_1, w2_1, b2_1,
           w1_2, b1_2, w2_2, b2_2, wp1, bp1, wp2, bp2, t_table):
    src = edge_index[0]
    dst = edge_index[1]
    zeros = jnp.zeros((N, D), jnp.float32)

    h = x
    for (w1, b1, w2, b2) in ((w1_0, b1_0, w2_0, b2_0),
                             (w1_1, b1_1, w2_1, b2_1),
                             (w1_2, b1_2, w2_2, b2_2)):
        parts = _agg_call(h, src, dst, zeros)
        h = _mlp(h, parts[:N], parts[N:], w1, b1, w2, b2)

    gidx = jnp.concatenate([
        candidate_edges[:, 0], candidate_edges[:, 1], first_edge,
        jnp.zeros((CB - 2 * 256 - 2,), jnp.int32),
    ]).astype(jnp.int32)
    gat = _gather_call(h, gidx)

    tt = jnp.clip(t, 0, t_table.shape[0] - 1).astype(jnp.int32).reshape(1)
    scores = _score(tt, gat, t_table, wp1, bp1, wp2, bp2)
    return scores[:, 0]
